# baseline (speedup 1.0000x reference)
import functools

import jax
import jax.numpy as jnp
from jax import lax
from jax.experimental import pallas as pl
from jax.experimental.pallas import tpu as pltpu
from jax.experimental.pallas import tpu_sc as plsc

N = 10000
E = 320000
D = 128
H = 64

NC = 2
NS = 16
NW = NC * NS
WIN = 128
NROWS = E // WIN
NWIN = NROWS // NW
NXTRA = NROWS - NWIN * NW
N_PAD = 10016
RPT = N_PAD // NS
F2 = 8
FD = 16

_MESH = plsc.VectorSubcoreMesh(
    core_axis_name="c", subcore_axis_name="s", num_cores=NC, num_subcores=NS)


def _make_agg(F, nbuf, name):
  assert NWIN % nbuf == 0

  @functools.partial(
      pl.kernel,
      out_type=jax.ShapeDtypeStruct((NC, N_PAD, F), jnp.float32),
      mesh=_MESH,
      scratch_types=[
          pltpu.VMEM((NWIN + 1, WIN), jnp.int32),
          pltpu.VMEM((NWIN + 1, WIN), jnp.int32),
          [pltpu.VMEM((WIN, F), jnp.float32)] * nbuf,
          pltpu.VMEM_SHARED((N_PAD, F), jnp.float32),
          [pltpu.SemaphoreType.DMA] * nbuf,
          [pltpu.SemaphoreType.DMA] * nbuf,
      ],
      compiler_params=pltpu.CompilerParams(use_tc_tiling_on_sc=False),
      name=name,
  )
  def agg(y_hbm, srcw_hbm, dstw_hbm, z_hbm, out_hbm, src_i, dst_i, rows,
          acc_sh, gsem, ssem):
    c = lax.axis_index("c")
    s = lax.axis_index("s")
    wid = c * NS + s
    wbase = wid * NWIN
    pltpu.sync_copy(srcw_hbm.at[pl.ds(wbase, NWIN)], src_i.at[pl.ds(0, NWIN)])
    pltpu.sync_copy(dstw_hbm.at[pl.ds(wbase, NWIN)], dst_i.at[pl.ds(0, NWIN)])

    @pl.when(wid < NXTRA)
    def _():
      xrow = NWIN * NW + wid
      pltpu.sync_copy(srcw_hbm.at[pl.ds(xrow, 1)], src_i.at[pl.ds(NWIN, 1)])
      pltpu.sync_copy(dstw_hbm.at[pl.ds(xrow, 1)], dst_i.at[pl.ds(NWIN, 1)])

    pltpu.sync_copy(z_hbm, acc_sh.at[pl.ds(s * RPT, RPT)])
    plsc.subcore_barrier()

    for b in range(nbuf):
      pltpu.async_copy(y_hbm.at[src_i.at[b]], rows[b], gsem[b])

    def grp(i, carry):
      w = nbuf * i
      for b in range(nbuf):
        pltpu.make_async_copy(y_hbm.at[src_i.at[w + b]], rows[b],
                              gsem[b]).wait()
        pltpu.async_copy(rows[b], acc_sh.at[dst_i.at[w + b]], ssem[b],
                         add=True)
      for b in range(nbuf):
        wn = w + nbuf + b

        @pl.when(wn < NWIN)
        def _():
          pltpu.make_async_copy(rows[b], acc_sh.at[dst_i.at[0]],
                                ssem[b]).wait()
          pltpu.async_copy(y_hbm.at[src_i.at[wn]], rows[b], gsem[b])

      return carry

    lax.fori_loop(0, NWIN // nbuf, grp, 0)
    for b in range(nbuf):
      pltpu.make_async_copy(rows[b], acc_sh.at[dst_i.at[0]], ssem[b]).wait()

    @pl.when(wid < NXTRA)
    def _():
      pltpu.sync_copy(y_hbm.at[src_i.at[NWIN]], rows[0])
      pltpu.sync_copy(rows[0], acc_sh.at[dst_i.at[NWIN]], add=True)

    plsc.subcore_barrier()
    pltpu.sync_copy(acc_sh.at[pl.ds(s * RPT, RPT)],
                    out_hbm.at[c].at[pl.ds(s * RPT, RPT)])

  return agg


_DEG_NBUF = 6


@functools.partial(
    pl.kernel,
    out_type=jax.ShapeDtypeStruct((NC, N_PAD, F2), jnp.float32),
    mesh=_MESH,
    scratch_types=[
        pltpu.VMEM((NWIN + 1, WIN), jnp.int32),
        pltpu.VMEM((WIN, F2), jnp.float32),
        pltpu.VMEM_SHARED((N_PAD, F2), jnp.float32),
        pltpu.SemaphoreType.DMA,
    ],
    compiler_params=pltpu.CompilerParams(use_tc_tiling_on_sc=False),
    name="gcn_deg_sc",
)
def _deg_sc(dstw_hbm, ones_hbm, z_hbm, out_hbm, dst_i, ones_v, acc_sh, sem):
  c = lax.axis_index("c")
  s = lax.axis_index("s")
  wid = c * NS + s
  pltpu.sync_copy(dstw_hbm.at[pl.ds(wid * NWIN, NWIN)],
                  dst_i.at[pl.ds(0, NWIN)])

  @pl.when(wid < NXTRA)
  def _():
    pltpu.sync_copy(dstw_hbm.at[pl.ds(NWIN * NW + wid, 1)],
                    dst_i.at[pl.ds(NWIN, 1)])

  pltpu.sync_copy(ones_hbm, ones_v)
  pltpu.sync_copy(z_hbm, acc_sh.at[pl.ds(s * RPT, RPT)])
  plsc.subcore_barrier()

  def grp(g, carry):
    base = g * _DEG_NBUF
    for j in range(_DEG_NBUF):
      pltpu.async_copy(ones_v, acc_sh.at[dst_i.at[base + j]], sem, add=True)
    for j in range(_DEG_NBUF):
      pltpu.make_async_copy(ones_v, acc_sh.at[dst_i.at[base + j]], sem).wait()
    return carry

  lax.fori_loop(0, NWIN // _DEG_NBUF, grp, 0)

  @pl.when(wid < NXTRA)
  def _():
    pltpu.sync_copy(ones_v, acc_sh.at[dst_i.at[NWIN]], add=True)

  plsc.subcore_barrier()
  pltpu.sync_copy(acc_sh.at[pl.ds(s * RPT, RPT)],
                  out_hbm.at[c].at[pl.ds(s * RPT, RPT)])


_agg64 = _make_agg(H, 6, "gcn_agg64_sc")
_agg8 = _make_agg(F2, 6, "gcn_agg8_sc")

_RB = 2000
_GRID = (N // _RB,)


def _tcmm1_body(x_ref, w1_ref, xw_ref):
  xw_ref[...] = jnp.dot(x_ref[...], w1_ref[...],
                        preferred_element_type=jnp.float32)


_tcmm1 = pl.pallas_call(
    _tcmm1_body,
    grid=_GRID,
    in_specs=[
        pl.BlockSpec((_RB, D), lambda i: (i, 0)),
        pl.BlockSpec((D, H), lambda i: (0, 0)),
    ],
    out_specs=pl.BlockSpec((_RB, H), lambda i: (i, 0)),
    out_shape=jax.ShapeDtypeStruct((N, H), jnp.float32),
)


def _tc1_body(xw_ref, dp_ref, y1_ref, dinv_ref):
  deg = 1.0 + dp_ref[0] + dp_ref[1]
  dinv = lax.rsqrt(deg)
  d0 = dinv[:, 0:1]
  y1_ref[...] = xw_ref[...] * d0
  dinv_ref[...] = jnp.concatenate([dinv, dinv], axis=1)


_tc1 = pl.pallas_call(
    _tc1_body,
    grid=_GRID,
    in_specs=[
        pl.BlockSpec((_RB, H), lambda i: (i, 0)),
        pl.BlockSpec((NC, _RB, F2), lambda i: (0, i, 0)),
    ],
    out_specs=[
        pl.BlockSpec((_RB, H), lambda i: (i, 0)),
        pl.BlockSpec((_RB, FD), lambda i: (i, 0)),
    ],
    out_shape=[
        jax.ShapeDtypeStruct((N, H), jnp.float32),
        jax.ShapeDtypeStruct((N, FD), jnp.float32),
    ],
)


def _tc2_body(a1_ref, xw_ref, dinv_ref, b1_ref, w2_ref, hw2_ref, y2_ref):
  d0 = dinv_ref[:, 0:1]
  agg = a1_ref[0] + a1_ref[1]
  out1 = d0 * agg + (d0 * d0) * xw_ref[...] + b1_ref[...]
  h = jnp.maximum(out1, 0.0)
  hw2 = jnp.dot(h, w2_ref[...], preferred_element_type=jnp.float32)
  hw2_ref[...] = hw2
  y2_ref[...] = hw2 * d0


_tc2 = pl.pallas_call(
    _tc2_body,
    grid=_GRID,
    in_specs=[
        pl.BlockSpec((NC, _RB, H), lambda i: (0, i, 0)),
        pl.BlockSpec((_RB, H), lambda i: (i, 0)),
        pl.BlockSpec((_RB, FD), lambda i: (i, 0)),
        pl.BlockSpec((1, H), lambda i: (0, 0)),
        pl.BlockSpec((H, F2), lambda i: (0, 0)),
    ],
    out_specs=[
        pl.BlockSpec((_RB, F2), lambda i: (i, 0)),
        pl.BlockSpec((_RB, F2), lambda i: (i, 0)),
    ],
    out_shape=[
        jax.ShapeDtypeStruct((N, F2), jnp.float32),
        jax.ShapeDtypeStruct((N, F2), jnp.float32),
    ],
)


def _tc3_body(a2_ref, hw2_ref, dinv_ref, b2_ref, out_ref):
  d0 = dinv_ref[:, 0:1]
  agg = a2_ref[0] + a2_ref[1]
  full = d0 * agg + (d0 * d0) * hw2_ref[...] + b2_ref[...]
  out_ref[...] = full[:, :2]


_tc3 = pl.pallas_call(
    _tc3_body,
    grid=_GRID,
    in_specs=[
        pl.BlockSpec((NC, _RB, F2), lambda i: (0, i, 0)),
        pl.BlockSpec((_RB, F2), lambda i: (i, 0)),
        pl.BlockSpec((_RB, FD), lambda i: (i, 0)),
        pl.BlockSpec((1, F2), lambda i: (0, 0)),
    ],
    out_specs=pl.BlockSpec((_RB, 2), lambda i: (i, 0)),
    out_shape=jax.ShapeDtypeStruct((N, 2), jnp.float32),
)


def kernel(x, edge_index, W1, b1, W2, b2):
  ei = edge_index.astype(jnp.int32)
  src = ei[0].reshape(NROWS, WIN)
  dst = ei[1].reshape(NROWS, WIN)

  ones_w = jnp.ones((WIN, F2), jnp.float32)
  z8 = jnp.zeros((RPT, F2), jnp.float32)
  z64 = jnp.zeros((RPT, H), jnp.float32)

  dp = _deg_sc(dst, ones_w, z8)
  xw = _tcmm1(x, W1)
  y1, dinv = _tc1(xw, dp)
  a1 = _agg64(y1, src, dst, z64)
  b1r = b1.reshape(1, H)
  w2p = jnp.zeros((H, F2), jnp.float32).at[:, :2].set(W2)
  hw2, y2 = _tc2(a1, xw, dinv, b1r, w2p)
  a2 = _agg8(y2, src, dst, z8)
  b2p = jnp.zeros((1, F2), jnp.float32).at[0, :2].set(b2)
  return _tc3(a2, hw2, dinv, b2p)


# speedup vs baseline: 1.0450x; 1.0450x over previous
import functools

import jax
import jax.numpy as jnp
from jax import lax
from jax.experimental import pallas as pl
from jax.experimental.pallas import tpu as pltpu
from jax.experimental.pallas import tpu_sc as plsc

N = 10000
E = 320000
D = 128
H = 64

NC = 2
NS = 16
NW = NC * NS
WIN = 256
NROWS = E // WIN
NWIN = NROWS // NW
NXTRA = NROWS - NWIN * NW
N_PAD = 10016
RPT = N_PAD // NS
F2 = 8
FD = 16

_MESH = plsc.VectorSubcoreMesh(
    core_axis_name="c", subcore_axis_name="s", num_cores=NC, num_subcores=NS)


def _make_agg(F, nbuf, name):
  assert NWIN % nbuf == 0

  @functools.partial(
      pl.kernel,
      out_type=jax.ShapeDtypeStruct((NC, N_PAD, F), jnp.float32),
      mesh=_MESH,
      scratch_types=[
          pltpu.VMEM((NWIN + 1, WIN), jnp.int32),
          pltpu.VMEM((NWIN + 1, WIN), jnp.int32),
          [pltpu.VMEM((WIN, F), jnp.float32)] * nbuf,
          pltpu.VMEM_SHARED((N_PAD, F), jnp.float32),
          [pltpu.SemaphoreType.DMA] * nbuf,
          [pltpu.SemaphoreType.DMA] * nbuf,
      ],
      compiler_params=pltpu.CompilerParams(use_tc_tiling_on_sc=False),
      name=name,
  )
  def agg(y_hbm, ei_hbm, z_hbm, out_hbm, src_i, dst_i, rows,
          acc_sh, gsem, ssem):
    c = lax.axis_index("c")
    s = lax.axis_index("s")
    wid = c * NS + s
    wbase = wid * NWIN
    srcw_hbm = ei_hbm.at[0]
    dstw_hbm = ei_hbm.at[1]
    pltpu.sync_copy(srcw_hbm.at[pl.ds(wbase, NWIN)], src_i.at[pl.ds(0, NWIN)])
    pltpu.sync_copy(dstw_hbm.at[pl.ds(wbase, NWIN)], dst_i.at[pl.ds(0, NWIN)])

    @pl.when(wid < NXTRA)
    def _():
      xrow = NWIN * NW + wid
      pltpu.sync_copy(srcw_hbm.at[pl.ds(xrow, 1)], src_i.at[pl.ds(NWIN, 1)])
      pltpu.sync_copy(dstw_hbm.at[pl.ds(xrow, 1)], dst_i.at[pl.ds(NWIN, 1)])

    pltpu.sync_copy(z_hbm, acc_sh.at[pl.ds(s * RPT, RPT)])
    plsc.subcore_barrier()

    for b in range(nbuf):
      pltpu.async_copy(y_hbm.at[src_i.at[b]], rows[b], gsem[b])

    def grp(i, carry):
      w = nbuf * i
      for b in range(nbuf):
        pltpu.make_async_copy(y_hbm.at[src_i.at[w + b]], rows[b],
                              gsem[b]).wait()
        pltpu.async_copy(rows[b], acc_sh.at[dst_i.at[w + b]], ssem[b],
                         add=True)
      for b in range(nbuf):
        wn = w + nbuf + b

        @pl.when(wn < NWIN)
        def _():
          pltpu.make_async_copy(rows[b], acc_sh.at[dst_i.at[0]],
                                ssem[b]).wait()
          pltpu.async_copy(y_hbm.at[src_i.at[wn]], rows[b], gsem[b])

      return carry

    lax.fori_loop(0, NWIN // nbuf, grp, 0)
    for b in range(nbuf):
      pltpu.make_async_copy(rows[b], acc_sh.at[dst_i.at[0]], ssem[b]).wait()

    @pl.when(wid < NXTRA)
    def _():
      pltpu.sync_copy(y_hbm.at[src_i.at[NWIN]], rows[0])
      pltpu.sync_copy(rows[0], acc_sh.at[dst_i.at[NWIN]], add=True)

    plsc.subcore_barrier()
    pltpu.sync_copy(acc_sh.at[pl.ds(s * RPT, RPT)],
                    out_hbm.at[c].at[pl.ds(s * RPT, RPT)])

  return agg


_DEG_NBUF = 3


@functools.partial(
    pl.kernel,
    out_type=jax.ShapeDtypeStruct((NC, N_PAD, F2), jnp.float32),
    mesh=_MESH,
    scratch_types=[
        pltpu.VMEM((NWIN + 1, WIN), jnp.int32),
        pltpu.VMEM((WIN, F2), jnp.float32),
        pltpu.VMEM_SHARED((N_PAD, F2), jnp.float32),
        pltpu.SemaphoreType.DMA,
    ],
    compiler_params=pltpu.CompilerParams(use_tc_tiling_on_sc=False),
    name="gcn_deg_sc",
)
def _deg_sc(ei_hbm, ones_hbm, z_hbm, out_hbm, dst_i, ones_v, acc_sh, sem):
  c = lax.axis_index("c")
  s = lax.axis_index("s")
  wid = c * NS + s
  dstw_hbm = ei_hbm.at[1]
  pltpu.sync_copy(dstw_hbm.at[pl.ds(wid * NWIN, NWIN)],
                  dst_i.at[pl.ds(0, NWIN)])

  @pl.when(wid < NXTRA)
  def _():
    pltpu.sync_copy(dstw_hbm.at[pl.ds(NWIN * NW + wid, 1)],
                    dst_i.at[pl.ds(NWIN, 1)])

  pltpu.sync_copy(ones_hbm, ones_v)
  pltpu.sync_copy(z_hbm, acc_sh.at[pl.ds(s * RPT, RPT)])
  plsc.subcore_barrier()

  def grp(g, carry):
    base = g * _DEG_NBUF
    for j in range(_DEG_NBUF):
      pltpu.async_copy(ones_v, acc_sh.at[dst_i.at[base + j]], sem, add=True)
    for j in range(_DEG_NBUF):
      pltpu.make_async_copy(ones_v, acc_sh.at[dst_i.at[base + j]], sem).wait()
    return carry

  lax.fori_loop(0, NWIN // _DEG_NBUF, grp, 0)

  @pl.when(wid < NXTRA)
  def _():
    pltpu.sync_copy(ones_v, acc_sh.at[dst_i.at[NWIN]], add=True)

  plsc.subcore_barrier()
  pltpu.sync_copy(acc_sh.at[pl.ds(s * RPT, RPT)],
                  out_hbm.at[c].at[pl.ds(s * RPT, RPT)])


_agg64 = _make_agg(H, 3, "gcn_agg64_sc")
_agg8 = _make_agg(F2, 3, "gcn_agg8_sc")

_RB = 5000
_GRID = (N // _RB,)


def _tcmm1_body(x_ref, w1_ref, xw_ref):
  xw_ref[...] = jnp.dot(x_ref[...], w1_ref[...],
                        preferred_element_type=jnp.float32)


_tcmm1 = pl.pallas_call(
    _tcmm1_body,
    grid=_GRID,
    in_specs=[
        pl.BlockSpec((_RB, D), lambda i: (i, 0)),
        pl.BlockSpec((D, H), lambda i: (0, 0)),
    ],
    out_specs=pl.BlockSpec((_RB, H), lambda i: (i, 0)),
    out_shape=jax.ShapeDtypeStruct((N, H), jnp.float32),
)


def _tc1_body(xw_ref, dp_ref, y1_ref, dinv_ref):
  deg = 1.0 + dp_ref[0] + dp_ref[1]
  dinv = lax.rsqrt(deg)
  d0 = dinv[:, 0:1]
  y1_ref[...] = xw_ref[...] * d0
  dinv_ref[...] = jnp.concatenate([dinv, dinv], axis=1)


_tc1 = pl.pallas_call(
    _tc1_body,
    grid=_GRID,
    in_specs=[
        pl.BlockSpec((_RB, H), lambda i: (i, 0)),
        pl.BlockSpec((NC, _RB, F2), lambda i: (0, i, 0)),
    ],
    out_specs=[
        pl.BlockSpec((_RB, H), lambda i: (i, 0)),
        pl.BlockSpec((_RB, FD), lambda i: (i, 0)),
    ],
    out_shape=[
        jax.ShapeDtypeStruct((N, H), jnp.float32),
        jax.ShapeDtypeStruct((N, FD), jnp.float32),
    ],
)


def _tc2_body(a1_ref, xw_ref, dinv_ref, b1_ref, w2_ref, hw2_ref, y2_ref):
  d0 = dinv_ref[:, 0:1]
  agg = a1_ref[0] + a1_ref[1]
  out1 = d0 * agg + (d0 * d0) * xw_ref[...] + b1_ref[...]
  h = jnp.maximum(out1, 0.0)
  hw2 = jnp.dot(h, w2_ref[...], preferred_element_type=jnp.float32)
  hw2_ref[...] = hw2
  y2_ref[...] = hw2 * d0


_tc2 = pl.pallas_call(
    _tc2_body,
    grid=_GRID,
    in_specs=[
        pl.BlockSpec((NC, _RB, H), lambda i: (0, i, 0)),
        pl.BlockSpec((_RB, H), lambda i: (i, 0)),
        pl.BlockSpec((_RB, FD), lambda i: (i, 0)),
        pl.BlockSpec((1, H), lambda i: (0, 0)),
        pl.BlockSpec((H, F2), lambda i: (0, 0)),
    ],
    out_specs=[
        pl.BlockSpec((_RB, F2), lambda i: (i, 0)),
        pl.BlockSpec((_RB, F2), lambda i: (i, 0)),
    ],
    out_shape=[
        jax.ShapeDtypeStruct((N, F2), jnp.float32),
        jax.ShapeDtypeStruct((N, F2), jnp.float32),
    ],
)


def _tc3_body(a2_ref, hw2_ref, dinv_ref, b2_ref, out_ref):
  d0 = dinv_ref[:, 0:1]
  agg = a2_ref[0] + a2_ref[1]
  full = d0 * agg + (d0 * d0) * hw2_ref[...] + b2_ref[...]
  out_ref[...] = full[:, :2]


_tc3 = pl.pallas_call(
    _tc3_body,
    grid=_GRID,
    in_specs=[
        pl.BlockSpec((NC, _RB, F2), lambda i: (0, i, 0)),
        pl.BlockSpec((_RB, F2), lambda i: (i, 0)),
        pl.BlockSpec((_RB, FD), lambda i: (i, 0)),
        pl.BlockSpec((1, F2), lambda i: (0, 0)),
    ],
    out_specs=pl.BlockSpec((_RB, 2), lambda i: (i, 0)),
    out_shape=jax.ShapeDtypeStruct((N, 2), jnp.float32),
)


def kernel(x, edge_index, W1, b1, W2, b2):
  ei3 = edge_index.astype(jnp.int32).reshape(2, NROWS, WIN)

  ones_w = jnp.ones((WIN, F2), jnp.float32)
  z8 = jnp.zeros((RPT, F2), jnp.float32)
  z64 = jnp.zeros((RPT, H), jnp.float32)

  dp = _deg_sc(ei3, ones_w, z8)
  xw = _tcmm1(x, W1)
  y1, dinv = _tc1(xw, dp)
  a1 = _agg64(y1, ei3, z64)
  b1r = b1.reshape(1, H)
  w2p = jnp.zeros((H, F2), jnp.float32).at[:, :2].set(W2)
  hw2, y2 = _tc2(a1, xw, dinv, b1r, w2p)
  a2 = _agg8(y2, ei3, z8)
  b2p = jnp.zeros((1, F2), jnp.float32).at[0, :2].set(b2)
  return _tc3(a2, hw2, dinv, b2p)


# speedup vs baseline: 1.0697x; 1.0237x over previous
import functools

import jax
import jax.numpy as jnp
from jax import lax
from jax.experimental import pallas as pl
from jax.experimental.pallas import tpu as pltpu
from jax.experimental.pallas import tpu_sc as plsc

N = 10000
E = 320000
D = 128
H = 64

NC = 2
NS = 16
NW = NC * NS
WIN = 256
NROWS = E // WIN
NWIN = NROWS // NW
NXTRA = NROWS - NWIN * NW
N_PAD = 10016
RPT = N_PAD // NS
F2 = 8
FD = 16

_MESH = plsc.VectorSubcoreMesh(
    core_axis_name="c", subcore_axis_name="s", num_cores=NC, num_subcores=NS)


def _make_agg(F, nbuf, sub, name):
  LW = WIN // sub
  NL = NWIN * sub

  @functools.partial(
      pl.kernel,
      out_type=jax.ShapeDtypeStruct((NC, N_PAD, F), jnp.float32),
      mesh=_MESH,
      scratch_types=[
          pltpu.VMEM((NWIN + 1, WIN), jnp.int32),
          pltpu.VMEM((NWIN + 1, WIN), jnp.int32),
          [pltpu.VMEM((LW, F), jnp.float32)] * nbuf,
          pltpu.VMEM_SHARED((N_PAD, F), jnp.float32),
          [pltpu.SemaphoreType.DMA] * nbuf,
          [pltpu.SemaphoreType.DMA] * nbuf,
      ],
      compiler_params=pltpu.CompilerParams(use_tc_tiling_on_sc=False),
      name=name,
  )
  def agg(y_hbm, ei_hbm, z_hbm, out_hbm, src_i, dst_i, rows,
          acc_sh, gsem, ssem):
    c = lax.axis_index("c")
    s = lax.axis_index("s")
    wid = c * NS + s
    wbase = wid * NWIN
    srcw_hbm = ei_hbm.at[0]
    dstw_hbm = ei_hbm.at[1]
    pltpu.sync_copy(srcw_hbm.at[pl.ds(wbase, NWIN)], src_i.at[pl.ds(0, NWIN)])
    pltpu.sync_copy(dstw_hbm.at[pl.ds(wbase, NWIN)], dst_i.at[pl.ds(0, NWIN)])

    @pl.when(wid < NXTRA)
    def _():
      xrow = NWIN * NW + wid
      pltpu.sync_copy(srcw_hbm.at[pl.ds(xrow, 1)], src_i.at[pl.ds(NWIN, 1)])
      pltpu.sync_copy(dstw_hbm.at[pl.ds(xrow, 1)], dst_i.at[pl.ds(NWIN, 1)])

    pltpu.sync_copy(z_hbm, acc_sh.at[pl.ds(s * RPT, RPT)])
    plsc.subcore_barrier()

    def sidx(ref, lw):
      return ref.at[lw // sub].at[pl.ds((lw % sub) * LW, LW)]

    for b in range(nbuf):
      pltpu.async_copy(y_hbm.at[sidx(src_i, b)], rows[b], gsem[b])

    ngrp = -(-NL // nbuf)

    def grp(i, carry):
      w = nbuf * i
      for b in range(nbuf):

        @pl.when(w + b < NL)
        def _():
          pltpu.make_async_copy(y_hbm.at[sidx(src_i, w + b)], rows[b],
                                gsem[b]).wait()
          pltpu.async_copy(rows[b], acc_sh.at[sidx(dst_i, w + b)], ssem[b],
                           add=True)

      for b in range(nbuf):
        wn = w + nbuf + b

        @pl.when(wn < NL)
        def _():
          pltpu.make_async_copy(rows[b], acc_sh.at[sidx(dst_i, 0)],
                                ssem[b]).wait()
          pltpu.async_copy(y_hbm.at[sidx(src_i, wn)], rows[b], gsem[b])

      return carry

    lax.fori_loop(0, ngrp, grp, 0)
    for b in range(nbuf):

      @pl.when(b < NL)
      def _():
        pltpu.make_async_copy(rows[b], acc_sh.at[sidx(dst_i, 0)],
                              ssem[b]).wait()

    @pl.when(wid < NXTRA)
    def _():
      for x in range(sub):
        pltpu.sync_copy(y_hbm.at[sidx(src_i, NWIN * sub + x)], rows[0])
        pltpu.sync_copy(rows[0], acc_sh.at[sidx(dst_i, NWIN * sub + x)],
                        add=True)

    plsc.subcore_barrier()
    pltpu.sync_copy(acc_sh.at[pl.ds(s * RPT, RPT)],
                    out_hbm.at[c].at[pl.ds(s * RPT, RPT)])

  return agg


_DEG_NBUF = 8


@functools.partial(
    pl.kernel,
    out_type=jax.ShapeDtypeStruct((NC, N_PAD, F2), jnp.float32),
    mesh=_MESH,
    scratch_types=[
        pltpu.VMEM((NWIN + 1, WIN), jnp.int32),
        pltpu.VMEM((WIN, F2), jnp.float32),
        pltpu.VMEM_SHARED((N_PAD, F2), jnp.float32),
        pltpu.SemaphoreType.DMA,
    ],
    compiler_params=pltpu.CompilerParams(use_tc_tiling_on_sc=False),
    name="gcn_deg_sc",
)
def _deg_sc(ei_hbm, ones_hbm, z_hbm, out_hbm, dst_i, ones_v, acc_sh, sem):
  c = lax.axis_index("c")
  s = lax.axis_index("s")
  wid = c * NS + s
  dstw_hbm = ei_hbm.at[1]
  pltpu.sync_copy(dstw_hbm.at[pl.ds(wid * NWIN, NWIN)],
                  dst_i.at[pl.ds(0, NWIN)])

  @pl.when(wid < NXTRA)
  def _():
    pltpu.sync_copy(dstw_hbm.at[pl.ds(NWIN * NW + wid, 1)],
                    dst_i.at[pl.ds(NWIN, 1)])

  pltpu.sync_copy(ones_hbm, ones_v)
  pltpu.sync_copy(z_hbm, acc_sh.at[pl.ds(s * RPT, RPT)])
  plsc.subcore_barrier()

  def grp(g, carry):
    base = g * _DEG_NBUF
    for j in range(_DEG_NBUF):

      @pl.when(base + j < NWIN)
      def _():
        pltpu.async_copy(ones_v, acc_sh.at[dst_i.at[base + j]], sem, add=True)

    for j in range(_DEG_NBUF):

      @pl.when(base + j < NWIN)
      def _():
        pltpu.make_async_copy(ones_v, acc_sh.at[dst_i.at[base + j]],
                              sem).wait()

    return carry

  lax.fori_loop(0, -(-NWIN // _DEG_NBUF), grp, 0)

  @pl.when(wid < NXTRA)
  def _():
    pltpu.sync_copy(ones_v, acc_sh.at[dst_i.at[NWIN]], add=True)

  plsc.subcore_barrier()
  pltpu.sync_copy(acc_sh.at[pl.ds(s * RPT, RPT)],
                  out_hbm.at[c].at[pl.ds(s * RPT, RPT)])


_agg64 = _make_agg(H, 6, 2, "gcn_agg64_sc")
_agg8 = _make_agg(F2, 8, 1, "gcn_agg8_sc")

_RB = 10000
_GRID = (N // _RB,)


def _tcmm1_body(x_ref, w1_ref, xw_ref):
  xw_ref[...] = jnp.dot(x_ref[...], w1_ref[...],
                        preferred_element_type=jnp.float32)


_tcmm1 = pl.pallas_call(
    _tcmm1_body,
    grid=_GRID,
    in_specs=[
        pl.BlockSpec((_RB, D), lambda i: (i, 0)),
        pl.BlockSpec((D, H), lambda i: (0, 0)),
    ],
    out_specs=pl.BlockSpec((_RB, H), lambda i: (i, 0)),
    out_shape=jax.ShapeDtypeStruct((N, H), jnp.float32),
)


def _tc1_body(xw_ref, dp_ref, y1_ref, dinv_ref):
  deg = 1.0 + dp_ref[0] + dp_ref[1]
  dinv = lax.rsqrt(deg)
  d0 = dinv[:, 0:1]
  y1_ref[...] = xw_ref[...] * d0
  dinv_ref[...] = jnp.concatenate([dinv, dinv], axis=1)


_tc1 = pl.pallas_call(
    _tc1_body,
    grid=_GRID,
    in_specs=[
        pl.BlockSpec((_RB, H), lambda i: (i, 0)),
        pl.BlockSpec((NC, _RB, F2), lambda i: (0, i, 0)),
    ],
    out_specs=[
        pl.BlockSpec((_RB, H), lambda i: (i, 0)),
        pl.BlockSpec((_RB, FD), lambda i: (i, 0)),
    ],
    out_shape=[
        jax.ShapeDtypeStruct((N, H), jnp.float32),
        jax.ShapeDtypeStruct((N, FD), jnp.float32),
    ],
)


def _tc2_body(a1_ref, xw_ref, dinv_ref, b1_ref, w2_ref, hw2_ref, y2_ref):
  d0 = dinv_ref[:, 0:1]
  agg = a1_ref[0] + a1_ref[1]
  out1 = d0 * agg + (d0 * d0) * xw_ref[...] + b1_ref[...]
  h = jnp.maximum(out1, 0.0)
  hw2 = jnp.dot(h, w2_ref[...], preferred_element_type=jnp.float32)
  hw2_ref[...] = hw2
  y2_ref[...] = hw2 * d0


_tc2 = pl.pallas_call(
    _tc2_body,
    grid=_GRID,
    in_specs=[
        pl.BlockSpec((NC, _RB, H), lambda i: (0, i, 0)),
        pl.BlockSpec((_RB, H), lambda i: (i, 0)),
        pl.BlockSpec((_RB, FD), lambda i: (i, 0)),
        pl.BlockSpec((1, H), lambda i: (0, 0)),
        pl.BlockSpec((H, F2), lambda i: (0, 0)),
    ],
    out_specs=[
        pl.BlockSpec((_RB, F2), lambda i: (i, 0)),
        pl.BlockSpec((_RB, F2), lambda i: (i, 0)),
    ],
    out_shape=[
        jax.ShapeDtypeStruct((N, F2), jnp.float32),
        jax.ShapeDtypeStruct((N, F2), jnp.float32),
    ],
)


def _tc3_body(a2_ref, hw2_ref, dinv_ref, b2_ref, out_ref):
  d0 = dinv_ref[:, 0:1]
  agg = a2_ref[0] + a2_ref[1]
  full = d0 * agg + (d0 * d0) * hw2_ref[...] + b2_ref[...]
  out_ref[...] = full[:, :2]


_tc3 = pl.pallas_call(
    _tc3_body,
    grid=_GRID,
    in_specs=[
        pl.BlockSpec((NC, _RB, F2), lambda i: (0, i, 0)),
        pl.BlockSpec((_RB, F2), lambda i: (i, 0)),
        pl.BlockSpec((_RB, FD), lambda i: (i, 0)),
        pl.BlockSpec((1, F2), lambda i: (0, 0)),
    ],
    out_specs=pl.BlockSpec((_RB, 2), lambda i: (i, 0)),
    out_shape=jax.ShapeDtypeStruct((N, 2), jnp.float32),
)


def kernel(x, edge_index, W1, b1, W2, b2):
  ei3 = edge_index.astype(jnp.int32).reshape(2, NROWS, WIN)

  ones_w = jnp.ones((WIN, F2), jnp.float32)
  z8 = jnp.zeros((RPT, F2), jnp.float32)
  z64 = jnp.zeros((RPT, H), jnp.float32)

  dp = _deg_sc(ei3, ones_w, z8)
  xw = _tcmm1(x, W1)
  y1, dinv = _tc1(xw, dp)
  a1 = _agg64(y1, ei3, z64)
  b1r = b1.reshape(1, H)
  w2p = jnp.zeros((H, F2), jnp.float32).at[:, :2].set(W2)
  hw2, y2 = _tc2(a1, xw, dinv, b1r, w2p)
  a2 = _agg8(y2, ei3, z8)
  b2p = jnp.zeros((1, F2), jnp.float32).at[0, :2].set(b2)
  return _tc3(a2, hw2, dinv, b2p)


# speedup vs baseline: 1.1041x; 1.0322x over previous
import functools

import jax
import jax.numpy as jnp
from jax import lax
from jax.experimental import pallas as pl
from jax.experimental.pallas import tpu as pltpu
from jax.experimental.pallas import tpu_sc as plsc

N = 10000
E = 320000
D = 128
H = 64

NC = 2
NS = 16
NW = NC * NS
WIN = 256
NROWS = E // WIN
NWIN = NROWS // NW
NXTRA = NROWS - NWIN * NW
N_PAD = 10016
RPT = N_PAD // NS
F2 = 8
FD = 16

_MESH = plsc.VectorSubcoreMesh(
    core_axis_name="c", subcore_axis_name="s", num_cores=NC, num_subcores=NS)


def _make_agg(F, nbuf, sub, name):
  LW = WIN // sub
  NL = NWIN * sub

  @functools.partial(
      pl.kernel,
      out_type=jax.ShapeDtypeStruct((NC, N_PAD, F), jnp.float32),
      mesh=_MESH,
      scratch_types=[
          pltpu.VMEM((NWIN + 1, WIN), jnp.int32),
          pltpu.VMEM((NWIN + 1, WIN), jnp.int32),
          [pltpu.VMEM((LW, F), jnp.float32)] * nbuf,
          pltpu.VMEM_SHARED((N_PAD, F), jnp.float32),
          [pltpu.SemaphoreType.DMA] * nbuf,
          [pltpu.SemaphoreType.DMA] * nbuf,
      ],
      compiler_params=pltpu.CompilerParams(use_tc_tiling_on_sc=False),
      name=name,
  )
  def agg(y_hbm, ei_hbm, z_hbm, out_hbm, src_i, dst_i, rows,
          acc_sh, gsem, ssem):
    c = lax.axis_index("c")
    s = lax.axis_index("s")
    wid = c * NS + s
    wbase = wid * NWIN
    srcw_hbm = ei_hbm.at[0]
    dstw_hbm = ei_hbm.at[1]
    pltpu.sync_copy(srcw_hbm.at[pl.ds(wbase, NWIN)], src_i.at[pl.ds(0, NWIN)])
    pltpu.sync_copy(dstw_hbm.at[pl.ds(wbase, NWIN)], dst_i.at[pl.ds(0, NWIN)])

    @pl.when(wid < NXTRA)
    def _():
      xrow = NWIN * NW + wid
      pltpu.sync_copy(srcw_hbm.at[pl.ds(xrow, 1)], src_i.at[pl.ds(NWIN, 1)])
      pltpu.sync_copy(dstw_hbm.at[pl.ds(xrow, 1)], dst_i.at[pl.ds(NWIN, 1)])

    pltpu.sync_copy(z_hbm, acc_sh.at[pl.ds(s * RPT, RPT)])
    plsc.subcore_barrier()

    def sidx(ref, lw):
      return ref.at[lw // sub].at[pl.ds((lw % sub) * LW, LW)]

    for b in range(nbuf):
      pltpu.async_copy(y_hbm.at[sidx(src_i, b)], rows[b], gsem[b])

    ngrp = -(-NL // nbuf)

    def grp(i, carry):
      w = nbuf * i
      for b in range(nbuf):

        @pl.when(w + b < NL)
        def _():
          pltpu.make_async_copy(y_hbm.at[sidx(src_i, w + b)], rows[b],
                                gsem[b]).wait()
          pltpu.async_copy(rows[b], acc_sh.at[sidx(dst_i, w + b)], ssem[b],
                           add=True)

      for b in range(nbuf):
        wn = w + nbuf + b

        @pl.when(wn < NL)
        def _():
          pltpu.make_async_copy(rows[b], acc_sh.at[sidx(dst_i, 0)],
                                ssem[b]).wait()
          pltpu.async_copy(y_hbm.at[sidx(src_i, wn)], rows[b], gsem[b])

      return carry

    lax.fori_loop(0, ngrp, grp, 0)
    for b in range(nbuf):

      @pl.when(b < NL)
      def _():
        pltpu.make_async_copy(rows[b], acc_sh.at[sidx(dst_i, 0)],
                              ssem[b]).wait()

    @pl.when(wid < NXTRA)
    def _():
      for x in range(sub):
        pltpu.sync_copy(y_hbm.at[sidx(src_i, NWIN * sub + x)], rows[0])
        pltpu.sync_copy(rows[0], acc_sh.at[sidx(dst_i, NWIN * sub + x)],
                        add=True)

    plsc.subcore_barrier()
    pltpu.sync_copy(acc_sh.at[pl.ds(s * RPT, RPT)],
                    out_hbm.at[c].at[pl.ds(s * RPT, RPT)])

  return agg


_DEG_NBUF = 8


@functools.partial(
    pl.kernel,
    out_type=jax.ShapeDtypeStruct((NC, N_PAD, F2), jnp.float32),
    mesh=_MESH,
    scratch_types=[
        pltpu.VMEM((NWIN + 1, WIN), jnp.int32),
        pltpu.VMEM((WIN, F2), jnp.float32),
        pltpu.VMEM_SHARED((N_PAD, F2), jnp.float32),
        pltpu.SemaphoreType.DMA,
    ],
    compiler_params=pltpu.CompilerParams(use_tc_tiling_on_sc=False),
    name="gcn_deg_sc",
)
def _deg_sc(ei_hbm, ones_hbm, z_hbm, out_hbm, dst_i, ones_v, acc_sh, sem):
  c = lax.axis_index("c")
  s = lax.axis_index("s")
  wid = c * NS + s
  dstw_hbm = ei_hbm.at[1]
  pltpu.sync_copy(dstw_hbm.at[pl.ds(wid * NWIN, NWIN)],
                  dst_i.at[pl.ds(0, NWIN)])

  @pl.when(wid < NXTRA)
  def _():
    pltpu.sync_copy(dstw_hbm.at[pl.ds(NWIN * NW + wid, 1)],
                    dst_i.at[pl.ds(NWIN, 1)])

  pltpu.sync_copy(ones_hbm, ones_v)
  pltpu.sync_copy(z_hbm, acc_sh.at[pl.ds(s * RPT, RPT)])
  plsc.subcore_barrier()

  def grp(g, carry):
    base = g * _DEG_NBUF
    for j in range(_DEG_NBUF):

      @pl.when(base + j < NWIN)
      def _():
        pltpu.async_copy(ones_v, acc_sh.at[dst_i.at[base + j]], sem, add=True)

    for j in range(_DEG_NBUF):

      @pl.when(base + j < NWIN)
      def _():
        pltpu.make_async_copy(ones_v, acc_sh.at[dst_i.at[base + j]],
                              sem).wait()

    return carry

  lax.fori_loop(0, -(-NWIN // _DEG_NBUF), grp, 0)

  @pl.when(wid < NXTRA)
  def _():
    pltpu.sync_copy(ones_v, acc_sh.at[dst_i.at[NWIN]], add=True)

  plsc.subcore_barrier()
  pltpu.sync_copy(acc_sh.at[pl.ds(s * RPT, RPT)],
                  out_hbm.at[c].at[pl.ds(s * RPT, RPT)])


_agg64 = _make_agg(H, 6, 2, "gcn_agg64_sc")
_agg8 = _make_agg(F2, 8, 1, "gcn_agg8_sc")

_RB = 5000
_GRID = (N // _RB,)


def _tcmm1_body(x_ref, w1_ref, xw_ref):
  xw_ref[...] = jnp.dot(x_ref[...], w1_ref[...],
                        preferred_element_type=jnp.float32)


_tcmm1 = pl.pallas_call(
    _tcmm1_body,
    grid=_GRID,
    in_specs=[
        pl.BlockSpec((_RB, D), lambda i: (i, 0)),
        pl.BlockSpec((D, H), lambda i: (0, 0)),
    ],
    out_specs=pl.BlockSpec((_RB, H), lambda i: (i, 0)),
    out_shape=jax.ShapeDtypeStruct((N, H), jnp.float32),
)


def _tc1_body(xw_ref, dp_ref, y1_ref, dinv_ref):
  deg = 1.0 + dp_ref[0] + dp_ref[1]
  dinv = lax.rsqrt(deg)
  d0 = dinv[:, 0:1]
  y1_ref[...] = xw_ref[...] * d0
  dinv_ref[...] = jnp.concatenate([dinv, dinv], axis=1)


_tc1 = pl.pallas_call(
    _tc1_body,
    grid=_GRID,
    in_specs=[
        pl.BlockSpec((_RB, H), lambda i: (i, 0)),
        pl.BlockSpec((NC, _RB, F2), lambda i: (0, i, 0)),
    ],
    out_specs=[
        pl.BlockSpec((_RB, H), lambda i: (i, 0)),
        pl.BlockSpec((_RB, FD), lambda i: (i, 0)),
    ],
    out_shape=[
        jax.ShapeDtypeStruct((N, H), jnp.float32),
        jax.ShapeDtypeStruct((N, FD), jnp.float32),
    ],
)


def _tc2_body(a1_ref, xw_ref, dinv_ref, b1_ref, w2_ref, hw2_ref, y2_ref):
  d0 = dinv_ref[:, 0:1]
  agg = a1_ref[0] + a1_ref[1]
  out1 = d0 * agg + (d0 * d0) * xw_ref[...] + b1_ref[...]
  h = jnp.maximum(out1, 0.0)
  hw2 = jnp.dot(h, w2_ref[...], preferred_element_type=jnp.float32)
  hw2_ref[...] = hw2
  y2_ref[...] = hw2 * d0


_tc2 = pl.pallas_call(
    _tc2_body,
    grid=_GRID,
    in_specs=[
        pl.BlockSpec((NC, _RB, H), lambda i: (0, i, 0)),
        pl.BlockSpec((_RB, H), lambda i: (i, 0)),
        pl.BlockSpec((_RB, FD), lambda i: (i, 0)),
        pl.BlockSpec((1, H), lambda i: (0, 0)),
        pl.BlockSpec((H, F2), lambda i: (0, 0)),
    ],
    out_specs=[
        pl.BlockSpec((_RB, F2), lambda i: (i, 0)),
        pl.BlockSpec((_RB, F2), lambda i: (i, 0)),
    ],
    out_shape=[
        jax.ShapeDtypeStruct((N, F2), jnp.float32),
        jax.ShapeDtypeStruct((N, F2), jnp.float32),
    ],
)


def _tc3_body(a2_ref, hw2_ref, dinv_ref, b2_ref, out_ref):
  d0 = dinv_ref[:, 0:1]
  agg = a2_ref[0] + a2_ref[1]
  full = d0 * agg + (d0 * d0) * hw2_ref[...] + b2_ref[...]
  out_ref[...] = full[:, :2]


_tc3 = pl.pallas_call(
    _tc3_body,
    grid=_GRID,
    in_specs=[
        pl.BlockSpec((NC, _RB, F2), lambda i: (0, i, 0)),
        pl.BlockSpec((_RB, F2), lambda i: (i, 0)),
        pl.BlockSpec((_RB, FD), lambda i: (i, 0)),
        pl.BlockSpec((1, F2), lambda i: (0, 0)),
    ],
    out_specs=pl.BlockSpec((_RB, 2), lambda i: (i, 0)),
    out_shape=jax.ShapeDtypeStruct((N, 2), jnp.float32),
)


def kernel(x, edge_index, W1, b1, W2, b2):
  ei3 = edge_index.astype(jnp.int32).reshape(2, NROWS, WIN)

  ones_w = jnp.ones((WIN, F2), jnp.float32)
  z8 = jnp.zeros((RPT, F2), jnp.float32)
  z64 = jnp.zeros((RPT, H), jnp.float32)

  dp = _deg_sc(ei3, ones_w, z8)
  xw = _tcmm1(x, W1)
  y1, dinv = _tc1(xw, dp)
  a1 = _agg64(y1, ei3, z64)
  b1r = b1.reshape(1, H)
  w2p = jnp.zeros((H, F2), jnp.float32).at[:, :2].set(W2)
  hw2, y2 = _tc2(a1, xw, dinv, b1r, w2p)
  a2 = _agg8(y2, ei3, z8)
  b2p = jnp.zeros((1, F2), jnp.float32).at[0, :2].set(b2)
  return _tc3(a2, hw2, dinv, b2p)
